# baseline (speedup 1.0000x reference)
import functools

import jax
import jax.numpy as jnp
from jax import lax
from jax.experimental import pallas as pl
from jax.experimental.pallas import tpu as pltpu
from jax.experimental.pallas import tpu_sc as plsc

_CHUNK = 64


@functools.partial(jax.jit, static_argnames=("n", "d"))
def _embed_flat(tokens_flat, W_E, n, d):
    info = plsc.get_sparse_core_info()
    nw = info.num_cores * info.num_subcores
    n_per_w = n // nw
    n_chunks = n_per_w // _CHUNK
    mesh = plsc.VectorSubcoreMesh(core_axis_name="c", subcore_axis_name="s")

    @functools.partial(
        pl.kernel,
        mesh=mesh,
        out_type=jax.ShapeDtypeStruct((n, d), jnp.float32),
        scratch_types=[
            pltpu.VMEM((n_per_w,), jnp.int32),
            pltpu.VMEM((_CHUNK, d), jnp.float32),
            pltpu.VMEM((_CHUNK, d), jnp.float32),
            pltpu.SemaphoreType.DMA,
            pltpu.SemaphoreType.DMA,
            pltpu.SemaphoreType.DMA,
            pltpu.SemaphoreType.DMA,
        ],
    )
    def k(tok_hbm, table_hbm, out_hbm, idx_v, buf0, buf1, g0, g1, w0, w1):
        wid = lax.axis_index("s") * info.num_cores + lax.axis_index("c")
        base = wid * n_per_w
        pltpu.sync_copy(tok_hbm.at[pl.ds(base, n_per_w)], idx_v)
        bufs = (buf0, buf1)
        gsems = (g0, g1)
        wsems = (w0, w1)

        def start_gather(c):
            b = c % 2
            return pltpu.async_copy(
                table_hbm.at[idx_v.at[pl.ds(c * _CHUNK, _CHUNK)]],
                bufs[b],
                gsems[b],
            )

        writes = [None, None]
        cur = start_gather(0)
        for c in range(n_chunks):
            b = c % 2
            cur.wait()
            w = pltpu.async_copy(
                bufs[b], out_hbm.at[pl.ds(base + c * _CHUNK, _CHUNK)], wsems[b]
            )
            if c + 1 < n_chunks:
                if writes[1 - b] is not None:
                    writes[1 - b].wait()
                cur = start_gather(c + 1)
            writes[b] = w
        writes[0].wait()
        writes[1].wait()

    return k(tokens_flat, W_E)


def kernel(tokens, W_E):
    b, s = tokens.shape
    v, d = W_E.shape
    flat = tokens.reshape(b * s).astype(jnp.int32)
    out = _embed_flat(flat, W_E, b * s, d)
    return out.reshape(b, s, d)


# speedup vs baseline: 1.0266x; 1.0266x over previous
import functools

import jax
import jax.numpy as jnp
from jax import lax
from jax.experimental import pallas as pl
from jax.experimental.pallas import tpu as pltpu
from jax.experimental.pallas import tpu_sc as plsc

_CHUNK = 32
_NBUF = 4


@functools.partial(jax.jit, static_argnames=("n", "d"))
def _embed_flat(tokens_flat, W_E, n, d):
    info = plsc.get_sparse_core_info()
    nw = info.num_cores * info.num_subcores
    n_per_w = n // nw
    n_chunks = n_per_w // _CHUNK
    mesh = plsc.VectorSubcoreMesh(core_axis_name="c", subcore_axis_name="s")

    @functools.partial(
        pl.kernel,
        mesh=mesh,
        out_type=jax.ShapeDtypeStruct((n, d), jnp.float32),
        scratch_types=[
            pltpu.VMEM((n_per_w,), jnp.int32),
        ]
        + [pltpu.VMEM((_CHUNK, d), jnp.float32) for _ in range(_NBUF)]
        + [pltpu.SemaphoreType.DMA for _ in range(2 * _NBUF)],
    )
    def k(tok_hbm, table_hbm, out_hbm, idx_v, *rest):
        bufs = rest[:_NBUF]
        gsems = rest[_NBUF : 2 * _NBUF]
        wsems = rest[2 * _NBUF :]
        wid = lax.axis_index("s") * info.num_cores + lax.axis_index("c")
        base = wid * n_per_w
        pltpu.sync_copy(tok_hbm.at[pl.ds(base, n_per_w)], idx_v)

        def start_gather(c):
            b = c % _NBUF
            return pltpu.async_copy(
                table_hbm.at[idx_v.at[pl.ds(c * _CHUNK, _CHUNK)]],
                bufs[b],
                gsems[b],
            )

        gathers = {}
        writes = {}
        for c in range(min(_NBUF - 1, n_chunks)):
            gathers[c] = start_gather(c)
        for c in range(n_chunks):
            b = c % _NBUF
            gathers.pop(c).wait()
            writes[c] = pltpu.async_copy(
                bufs[b], out_hbm.at[pl.ds(base + c * _CHUNK, _CHUNK)], wsems[b]
            )
            nc = c + _NBUF - 1
            if nc < n_chunks:
                if c >= 1:
                    writes.pop(c - 1).wait()
                gathers[nc] = start_gather(nc)
        for c in sorted(writes):
            writes.pop(c).wait()

    return k(tokens_flat, W_E)


def kernel(tokens, W_E):
    b, s = tokens.shape
    v, d = W_E.shape
    flat = tokens.reshape(b * s).astype(jnp.int32)
    out = _embed_flat(flat, W_E, b * s, d)
    return out.reshape(b, s, d)
